# baseline (speedup 1.0000x reference)
import jax
import jax.numpy as jnp
from jax import lax
from jax.experimental import pallas as pl
from jax.experimental.pallas import tpu as pltpu
from jax.experimental.pallas import tpu_sc as plsc

DIM = 16
BATCH = 16384
NC = 2
NS = 16
L = 16
NW = NC * NS
BPW = BATCH // NW
CHUNK = 128
NCHUNK = (2 * BPW) // CHUNK
NBUF = 4
GPC = CHUNK // (2 * L)


def _loss_body(x_hbm, deg_hbm, tab_hbm, out_hbm, idx_v, idx8_v, deg_v,
               slab_v, res_v, *sems):
    wid = lax.axis_index("s") * NC + lax.axis_index("c")
    pltpu.sync_copy(x_hbm.at[wid], idx_v)
    pltpu.sync_copy(deg_hbm.at[wid], deg_v)

    for k in range(NCHUNK):
        for i in range(CHUNK // L):
            idx8_v[k, pl.ds(i * L, L)] = idx_v[k, pl.ds(i * L, L)] >> 3

    def fire(k, slot):
        return pltpu.async_copy(tab_hbm.at[idx8_v.at[k]], slab_v.at[slot],
                                sems[slot])

    copies = {}
    for k in range(NBUF):
        copies[k] = fire(k, k)

    iota = lax.iota(jnp.int32, L)

    def sqrt16(s):
        i = plsc.bitcast(s, jnp.int32)
        i = jnp.int32(0x5F3759DF) - (i >> 1)
        y = plsc.bitcast(i, jnp.float32)
        for _ in range(3):
            y = y * (1.5 - 0.5 * s * y * y)
        return jnp.where(s > 0.0, s * y, 0.0)

    acc = jnp.zeros((L,), jnp.float32)
    k_vec = [jnp.full((L,), k, jnp.int32) for k in range(NCHUNK)]
    for k in range(NCHUNK):
        copies[k].wait()
        slot = jnp.full((L,), k % NBUF, jnp.int32)
        for gl in range(GPC):
            p0 = gl * 2 * L + iota * 2
            p1 = p0 + 1
            v0 = plsc.load_gather(idx_v, [k_vec[k], p0])
            v1 = plsc.load_gather(idx_v, [k_vec[k], p1])
            off0 = (v0 & 7) * DIM
            off1 = (v1 & 7) * DIM
            s = jnp.zeros((L,), jnp.float32)
            for d in range(DIM):
                a = plsc.load_gather(slab_v, [slot, p0, off0 + d])
                b = plsc.load_gather(slab_v, [slot, p1, off1 + d])
                df = a - b
                s = s + df * df
            dist = sqrt16(s)
            g = k * GPC + gl
            deg = deg_v[pl.ds(g * L, L)]
            acc = acc + deg * dist + jnp.exp(-dist)
        if k + NBUF < NCHUNK:
            copies[k + NBUF] = fire(k + NBUF, k % NBUF)

    res_v[...] = jnp.full((L,), jnp.sum(acc), jnp.float32)
    pltpu.sync_copy(res_v, out_hbm.at[wid])


def kernel(x, degrees, embs):
    tab = embs.reshape(125000, 128)
    xr = x.astype(jnp.int32).reshape(NW, NCHUNK, CHUNK)
    dr = degrees.reshape(NW, BPW)
    mesh = plsc.VectorSubcoreMesh(core_axis_name="c", subcore_axis_name="s")
    out = pl.kernel(
        _loss_body,
        mesh=mesh,
        out_type=jax.ShapeDtypeStruct((NW, L), jnp.float32),
        scratch_types=[
            pltpu.VMEM((NCHUNK, CHUNK), jnp.int32),
            pltpu.VMEM((NCHUNK, CHUNK), jnp.int32),
            pltpu.VMEM((BPW,), jnp.float32),
            pltpu.VMEM((NBUF, CHUNK, 128), jnp.float32),
            pltpu.VMEM((L,), jnp.float32),
        ] + [pltpu.SemaphoreType.DMA] * NBUF,
        compiler_params=pltpu.CompilerParams(needs_layout_passes=False),
    )(xr, dr, tab)
    return jnp.sum(out[:, 0])
